# baseline (speedup 1.0000x reference)
import functools

import jax
import jax.numpy as jnp
from jax import lax
from jax.experimental import pallas as pl
from jax.experimental.pallas import tpu as pltpu
from jax.experimental.pallas import tpu_sc as plsc

N = 10000
NPAD = 10240
E = 320000
D_IN = 128
D_HID = 128
N_CLS = 8
N_CLS_PAD = 16

NW = 32
EPT = E // NW
KB = 125
NBLK = EPT // KB
CH = 16
NCH = NBLK // CH
APAD = 10112
TRASH = APAD - 1
RPT = APAD // 16

_ROWS = 1024
_GRID = (N + _ROWS - 1) // _ROWS

_mesh = plsc.VectorSubcoreMesh(core_axis_name="c", subcore_axis_name="s")
_sc_params = pltpu.CompilerParams(needs_layout_passes=False)


@functools.partial(
    pl.kernel,
    mesh=_mesh,
    out_type=jax.ShapeDtypeStruct((NW, NPAD), jnp.float32),
    compiler_params=_sc_params,
    scratch_types=[
        pltpu.VMEM((EPT,), jnp.int32),
        pltpu.VMEM((NPAD,), jnp.float32),
    ],
)
def _sc_deg(dst_hbm, out_hbm, dst_v, deg_v):
    c = lax.axis_index("c")
    s = lax.axis_index("s")
    w = c * 16 + s
    zero16 = jnp.zeros((16,), jnp.float32)

    def zbody(i, carry):
        deg_v[pl.ds(i * 16, 16)] = zero16
        return carry

    lax.fori_loop(0, NPAD // 16, zbody, 0)
    pltpu.sync_copy(dst_hbm.at[pl.ds(w * EPT, EPT)], dst_v)
    one16 = jnp.ones((16,), jnp.float32)

    def body(i, carry):
        idx = dst_v[pl.ds(i * 16, 16)]
        plsc.addupdate_scatter(deg_v, [idx], one16)
        return carry

    lax.fori_loop(0, EPT // 16, body, 0)
    pltpu.sync_copy(deg_v, out_hbm.at[w])


def _sc_scatter_body(src_hbm, dst_hbm, g_hbm, zeros_hbm, out_hbm,
                     src_ca, dst_ca, src_cb, dst_cb, stage_a, stage_b, acc,
                     gs_a, gs_b, ss_a, ss_b, is_a, is_b):
    c = lax.axis_index("c")
    s = lax.axis_index("s")
    pltpu.sync_copy(zeros_hbm.at[pl.ds(s * RPT, RPT)],
                    acc.at[pl.ds(s * RPT, RPT)])
    plsc.subcore_barrier()

    def load_chunk(ci, src_c, dst_c, sem):
        pltpu.async_copy(src_hbm.at[c, s, pl.ds(ci * CH, CH)], src_c, sem)
        pltpu.async_copy(dst_hbm.at[c, s, pl.ds(ci * CH, CH)], dst_c, sem)

    def wait_chunk(ci, src_c, dst_c, sem):
        pltpu.make_async_copy(
            src_hbm.at[c, s, pl.ds(ci * CH, CH)], src_c, sem).wait()
        pltpu.make_async_copy(
            dst_hbm.at[c, s, pl.ds(ci * CH, CH)], dst_c, sem).wait()

    def gather(src_c, j, stage, sem):
        pltpu.async_copy(g_hbm.at[src_c.at[j]], stage, sem)

    def wait_gather(src_c, j, stage, sem):
        pltpu.make_async_copy(g_hbm.at[src_c.at[j]], stage, sem).wait()

    def scat(dst_c, j, stage, sem):
        pltpu.async_copy(stage, acc.at[dst_c.at[j]], sem, add=True)

    def wait_scat(dst_c, j, stage, sem):
        pltpu.make_async_copy(stage, acc.at[dst_c.at[j]], sem).wait()

    def process_chunk(src_c, dst_c):
        gather(src_c, 0, stage_a, gs_a)

        def body(jj, carry):
            j0 = 2 * jj
            j1 = j0 + 1
            gather(src_c, j1, stage_b, gs_b)
            wait_gather(src_c, j0, stage_a, gs_a)
            scat(dst_c, j0, stage_a, ss_a)
            wait_scat(dst_c, j0, stage_a, ss_a)
            gather(src_c, j0 + 2, stage_a, gs_a)
            wait_gather(src_c, j1, stage_b, gs_b)
            scat(dst_c, j1, stage_b, ss_b)
            wait_scat(dst_c, j1, stage_b, ss_b)
            return carry

        lax.fori_loop(0, CH // 2 - 1, body, 0)
        gather(src_c, CH - 1, stage_b, gs_b)
        wait_gather(src_c, CH - 2, stage_a, gs_a)
        scat(dst_c, CH - 2, stage_a, ss_a)
        wait_scat(dst_c, CH - 2, stage_a, ss_a)
        wait_gather(src_c, CH - 1, stage_b, gs_b)
        scat(dst_c, CH - 1, stage_b, ss_b)
        wait_scat(dst_c, CH - 1, stage_b, ss_b)

    banks = [(src_ca, dst_ca, is_a), (src_cb, dst_cb, is_b)]
    load_chunk(0, *banks[0])
    for ci in range(NCH):
        bk = banks[ci % 2]
        if ci + 1 < NCH:
            load_chunk(ci + 1, *banks[(ci + 1) % 2])
        wait_chunk(ci, *bk)
        process_chunk(bk[0], bk[1])

    plsc.subcore_barrier()
    pltpu.sync_copy(acc.at[pl.ds(s * RPT, RPT)],
                    out_hbm.at[c, pl.ds(s * RPT, RPT)])


def _make_sc_scatter(d):
    return functools.partial(
        pl.kernel,
        mesh=_mesh,
        out_type=jax.ShapeDtypeStruct((2, APAD, d), jnp.float32),
        compiler_params=_sc_params,
        scratch_types=[
            pltpu.VMEM((CH, KB), jnp.int32),
            pltpu.VMEM((CH, KB), jnp.int32),
            pltpu.VMEM((CH, KB), jnp.int32),
            pltpu.VMEM((CH, KB), jnp.int32),
            pltpu.VMEM((KB, d), jnp.float32),
            pltpu.VMEM((KB, d), jnp.float32),
            pltpu.VMEM_SHARED((APAD, d), jnp.float32),
            pltpu.SemaphoreType.DMA,
            pltpu.SemaphoreType.DMA,
            pltpu.SemaphoreType.DMA,
            pltpu.SemaphoreType.DMA,
            pltpu.SemaphoreType.DMA,
            pltpu.SemaphoreType.DMA,
        ],
    )(_sc_scatter_body)


_sc_scatter_128 = _make_sc_scatter(D_HID)


def _mm_body(x_ref, w_ref, h_ref):
    h_ref[...] = jnp.dot(x_ref[...], w_ref[...],
                         preferred_element_type=jnp.float32)


def _dinv_body(degp_ref, dinv_ref):
    deg = jnp.sum(degp_ref[...], axis=0) + 1.0
    dinv_ref[...] = lax.rsqrt(deg)[:, None]


def _scale_body(degp_ref, h_ref, g_ref, dinv_ref):
    i = pl.program_id(0)
    part = degp_ref[:, pl.ds(i * _ROWS, _ROWS)]
    deg = jnp.sum(part, axis=0) + 1.0
    dinv = lax.rsqrt(deg)[:, None]
    g_ref[...] = h_ref[...] * dinv
    dinv_ref[...] = dinv


def _dinv_tc(deg_parts):
    return pl.pallas_call(
        _dinv_body,
        grid=(NPAD // 1280,),
        in_specs=[pl.BlockSpec((NW, 1280), lambda i: (0, i))],
        out_specs=pl.BlockSpec((1280, 1), lambda i: (i, 0)),
        out_shape=jax.ShapeDtypeStruct((NPAD, 1), jnp.float32),
    )(deg_parts)


def _scale2_body(h_ref, dinv_ref, g_ref):
    g_ref[...] = h_ref[...] * dinv_ref[...]


def _tc_scale2(h, dinv):
    d = h.shape[1]
    return pl.pallas_call(
        _scale2_body,
        grid=(_GRID,),
        in_specs=[_rows_spec(d), _rows_spec(1)],
        out_specs=_rows_spec(d),
        out_shape=jax.ShapeDtypeStruct((N, d), jnp.float32),
    )(h, dinv)


def _k2_body(s_ref, g_ref, dinv_ref, b_ref, gz_ref):
    agg = s_ref[0] + s_ref[1] + g_ref[...]
    z = jnp.maximum(agg * dinv_ref[...] + b_ref[...], 0.0)
    gz_ref[...] = z * dinv_ref[...]


def _k3_body(s_ref, g_ref, dinv_ref, b_ref, w2_ref, out_ref):
    agg = (s_ref[0] + s_ref[1] + g_ref[...]) * dinv_ref[...]
    out_ref[...] = (
        jnp.dot(agg, w2_ref[...], preferred_element_type=jnp.float32)
        + b_ref[...]
    )


def _rows_spec(d):
    return pl.BlockSpec((_ROWS, d), lambda i: (i, 0))


def _parts_spec(d):
    return pl.BlockSpec((2, _ROWS, d), lambda i: (0, i, 0))


def _full_spec(shape):
    return pl.BlockSpec(shape, lambda i: (0,) * len(shape))


def _tc_mm(x, w):
    return pl.pallas_call(
        _mm_body,
        grid=(_GRID,),
        in_specs=[_rows_spec(D_IN), _full_spec(w.shape)],
        out_specs=_rows_spec(w.shape[1]),
        out_shape=jax.ShapeDtypeStruct((N, w.shape[1]), jnp.float32),
    )(x, w)


def _tc_scale(deg_parts, h):
    d = h.shape[1]
    return pl.pallas_call(
        _scale_body,
        grid=(_GRID,),
        in_specs=[pl.BlockSpec((NW, NPAD), lambda i: (0, 0)),
                  _rows_spec(d)],
        out_specs=[_rows_spec(d), _rows_spec(1)],
        out_shape=[jax.ShapeDtypeStruct((N, d), jnp.float32),
                   jax.ShapeDtypeStruct((N, 1), jnp.float32)],
    )(deg_parts, h)


def _tc2(s_parts, g, dinv, b):
    d = g.shape[1]
    return pl.pallas_call(
        _k2_body,
        grid=(_GRID,),
        in_specs=[_parts_spec(d), _rows_spec(d), _rows_spec(1),
                  _full_spec((1, d))],
        out_specs=_rows_spec(d),
        out_shape=jax.ShapeDtypeStruct((N, d), jnp.float32),
    )(s_parts, g, dinv, b.reshape(1, d))


def _tc3(s_parts, g, dinv, b, w2):
    d = g.shape[1]
    return pl.pallas_call(
        _k3_body,
        grid=(_GRID,),
        in_specs=[_parts_spec(d), _rows_spec(d), _rows_spec(1),
                  _full_spec((1, N_CLS)), _full_spec(w2.shape)],
        out_specs=pl.BlockSpec((_ROWS, N_CLS), lambda i: (i, 0)),
        out_shape=jax.ShapeDtypeStruct((N, N_CLS), jnp.float32),
    )(s_parts, g, dinv, b.reshape(1, N_CLS), w2)


def kernel(x, edge_index, W1, b1, W2, b2):
    src = edge_index[0]
    dst = edge_index[1]
    srcr = src.reshape(2, 16, NBLK, KB)
    dstr = dst.reshape(2, 16, NBLK, KB)

    deg_parts = _sc_deg(dst)
    h1 = _tc_mm(x, W1)
    dinv = _dinv_tc(deg_parts)[:N]
    g1 = _tc_scale2(h1, dinv)
    s1p = _sc_scatter_128(srcr, dstr, g1, jnp.zeros((APAD, D_HID), jnp.float32))

    gz = _tc2(s1p, g1, dinv, b1)
    s2p = _sc_scatter_128(srcr, dstr, gz, jnp.zeros((APAD, D_HID), jnp.float32))
    return _tc3(s2p, gz, dinv, b2, W2)


# speedup vs baseline: 1.0211x; 1.0211x over previous
import functools

import jax
import jax.numpy as jnp
from jax import lax
from jax.experimental import pallas as pl
from jax.experimental.pallas import tpu as pltpu
from jax.experimental.pallas import tpu_sc as plsc

N = 10000
NPAD = 10240
E = 320000
D_IN = 128
D_HID = 128
N_CLS = 8
N_CLS_PAD = 16

NW = 32
EPT = E // NW
KB = 125
NBLK = EPT // KB
CH = 16
NCH = NBLK // CH
APAD = 10112
TRASH = APAD - 1
RPT = APAD // 16

_ROWS = 1024
_GRID = (N + _ROWS - 1) // _ROWS

_mesh = plsc.VectorSubcoreMesh(core_axis_name="c", subcore_axis_name="s")
_sc_params = pltpu.CompilerParams(needs_layout_passes=False)


@functools.partial(
    pl.kernel,
    mesh=_mesh,
    out_type=jax.ShapeDtypeStruct((NW, NPAD), jnp.float32),
    compiler_params=_sc_params,
    scratch_types=[
        pltpu.VMEM((EPT,), jnp.int32),
        pltpu.VMEM((NPAD,), jnp.float32),
    ],
)
def _sc_deg(dst_hbm, out_hbm, dst_v, deg_v):
    c = lax.axis_index("c")
    s = lax.axis_index("s")
    w = c * 16 + s
    zero16 = jnp.zeros((16,), jnp.float32)

    def zbody(i, carry):
        deg_v[pl.ds(i * 16, 16)] = zero16
        return carry

    lax.fori_loop(0, NPAD // 16, zbody, 0)
    pltpu.sync_copy(dst_hbm.at[pl.ds(w * EPT, EPT)], dst_v)
    one16 = jnp.ones((16,), jnp.float32)

    def body(i, carry):
        idx = dst_v[pl.ds(i * 16, 16)]
        plsc.addupdate_scatter(deg_v, [idx], one16)
        return carry

    lax.fori_loop(0, EPT // 16, body, 0)
    pltpu.sync_copy(deg_v, out_hbm.at[w])


def _sc_scatter_body(src_hbm, dst_hbm, g_hbm, zeros_hbm, out_hbm,
                     src_ca, dst_ca, src_cb, dst_cb, stage_a, stage_b, acc,
                     gs_a, gs_b, ss_a, ss_b, is_a, is_b):
    c = lax.axis_index("c")
    s = lax.axis_index("s")
    pltpu.sync_copy(zeros_hbm.at[pl.ds(s * RPT, RPT)],
                    acc.at[pl.ds(s * RPT, RPT)])
    plsc.subcore_barrier()

    def load_chunk(ci, src_c, dst_c, sem):
        pltpu.async_copy(src_hbm.at[c, s, pl.ds(ci * CH, CH)], src_c, sem)
        pltpu.async_copy(dst_hbm.at[c, s, pl.ds(ci * CH, CH)], dst_c, sem)

    def wait_chunk(ci, src_c, dst_c, sem):
        pltpu.make_async_copy(
            src_hbm.at[c, s, pl.ds(ci * CH, CH)], src_c, sem).wait()
        pltpu.make_async_copy(
            dst_hbm.at[c, s, pl.ds(ci * CH, CH)], dst_c, sem).wait()

    def gather(src_c, j, stage, sem):
        pltpu.async_copy(g_hbm.at[src_c.at[j]], stage, sem)

    def wait_gather(src_c, j, stage, sem):
        pltpu.make_async_copy(g_hbm.at[src_c.at[j]], stage, sem).wait()

    def scat(dst_c, j, stage, sem):
        pltpu.async_copy(stage, acc.at[dst_c.at[j]], sem, add=True)

    def wait_scat(dst_c, j, stage, sem):
        pltpu.make_async_copy(stage, acc.at[dst_c.at[j]], sem).wait()

    def process_chunk(src_c, dst_c):
        gather(src_c, 0, stage_a, gs_a)

        def body(jj, carry):
            j0 = 2 * jj
            j1 = j0 + 1
            gather(src_c, j1, stage_b, gs_b)
            wait_gather(src_c, j0, stage_a, gs_a)
            scat(dst_c, j0, stage_a, ss_a)
            wait_scat(dst_c, j0, stage_a, ss_a)
            gather(src_c, j0 + 2, stage_a, gs_a)
            wait_gather(src_c, j1, stage_b, gs_b)
            scat(dst_c, j1, stage_b, ss_b)
            wait_scat(dst_c, j1, stage_b, ss_b)
            return carry

        lax.fori_loop(0, CH // 2 - 1, body, 0)
        gather(src_c, CH - 1, stage_b, gs_b)
        wait_gather(src_c, CH - 2, stage_a, gs_a)
        scat(dst_c, CH - 2, stage_a, ss_a)
        wait_scat(dst_c, CH - 2, stage_a, ss_a)
        wait_gather(src_c, CH - 1, stage_b, gs_b)
        scat(dst_c, CH - 1, stage_b, ss_b)
        wait_scat(dst_c, CH - 1, stage_b, ss_b)

    banks = [(src_ca, dst_ca, is_a), (src_cb, dst_cb, is_b)]
    load_chunk(0, *banks[0])
    for ci in range(NCH):
        bk = banks[ci % 2]
        if ci + 1 < NCH:
            load_chunk(ci + 1, *banks[(ci + 1) % 2])
        wait_chunk(ci, *bk)
        process_chunk(bk[0], bk[1])

    plsc.subcore_barrier()
    pltpu.sync_copy(acc.at[pl.ds(s * RPT, RPT)],
                    out_hbm.at[c, pl.ds(s * RPT, RPT)])


def _make_sc_scatter(d):
    return functools.partial(
        pl.kernel,
        mesh=_mesh,
        out_type=jax.ShapeDtypeStruct((2, APAD, d), jnp.float32),
        compiler_params=_sc_params,
        scratch_types=[
            pltpu.VMEM((CH, KB), jnp.int32),
            pltpu.VMEM((CH, KB), jnp.int32),
            pltpu.VMEM((CH, KB), jnp.int32),
            pltpu.VMEM((CH, KB), jnp.int32),
            pltpu.VMEM((KB, d), jnp.float32),
            pltpu.VMEM((KB, d), jnp.float32),
            pltpu.VMEM_SHARED((APAD, d), jnp.float32),
            pltpu.SemaphoreType.DMA,
            pltpu.SemaphoreType.DMA,
            pltpu.SemaphoreType.DMA,
            pltpu.SemaphoreType.DMA,
            pltpu.SemaphoreType.DMA,
            pltpu.SemaphoreType.DMA,
        ],
    )(_sc_scatter_body)


_sc_scatter_128 = _make_sc_scatter(D_HID)


def _dinv_body(degp_ref, dinv_ref):
    deg = jnp.sum(degp_ref[...], axis=0) + 1.0
    dinv_ref[...] = lax.rsqrt(deg)[:, None]


def _dinv_tc(deg_parts):
    return pl.pallas_call(
        _dinv_body,
        grid=(NPAD // 1280,),
        in_specs=[pl.BlockSpec((NW, 1280), lambda i: (0, i))],
        out_specs=pl.BlockSpec((1280, 1), lambda i: (i, 0)),
        out_shape=jax.ShapeDtypeStruct((NPAD, 1), jnp.float32),
    )(deg_parts)


def _k1_body(x_ref, w_ref, dinv_ref, g_ref):
    h = jnp.dot(x_ref[...], w_ref[...], preferred_element_type=jnp.float32)
    g_ref[...] = h * dinv_ref[...]


def _tc1(x, w, dinv):
    return pl.pallas_call(
        _k1_body,
        grid=(_GRID,),
        in_specs=[_rows_spec(D_IN), _full_spec(w.shape), _rows_spec(1)],
        out_specs=_rows_spec(w.shape[1]),
        out_shape=jax.ShapeDtypeStruct((N, w.shape[1]), jnp.float32),
    )(x, w, dinv)


def _k2_body(s_ref, g_ref, dinv_ref, b_ref, gz_ref):
    agg = s_ref[0] + s_ref[1] + g_ref[...]
    z = jnp.maximum(agg * dinv_ref[...] + b_ref[...], 0.0)
    gz_ref[...] = z * dinv_ref[...]


def _k3_body(s_ref, g_ref, dinv_ref, b_ref, w2_ref, out_ref):
    agg = (s_ref[0] + s_ref[1] + g_ref[...]) * dinv_ref[...]
    out_ref[...] = (
        jnp.dot(agg, w2_ref[...], preferred_element_type=jnp.float32)
        + b_ref[...]
    )


def _rows_spec(d):
    return pl.BlockSpec((_ROWS, d), lambda i: (i, 0))


def _parts_spec(d):
    return pl.BlockSpec((2, _ROWS, d), lambda i: (0, i, 0))


def _full_spec(shape):
    return pl.BlockSpec(shape, lambda i: (0,) * len(shape))


def _tc2(s_parts, g, dinv, b):
    d = g.shape[1]
    return pl.pallas_call(
        _k2_body,
        grid=(_GRID,),
        in_specs=[_parts_spec(d), _rows_spec(d), _rows_spec(1),
                  _full_spec((1, d))],
        out_specs=_rows_spec(d),
        out_shape=jax.ShapeDtypeStruct((N, d), jnp.float32),
    )(s_parts, g, dinv, b.reshape(1, d))


def _tc3(s_parts, g, dinv, b, w2):
    d = g.shape[1]
    return pl.pallas_call(
        _k3_body,
        grid=(_GRID,),
        in_specs=[_parts_spec(d), _rows_spec(d), _rows_spec(1),
                  _full_spec((1, N_CLS)), _full_spec(w2.shape)],
        out_specs=pl.BlockSpec((_ROWS, N_CLS), lambda i: (i, 0)),
        out_shape=jax.ShapeDtypeStruct((N, N_CLS), jnp.float32),
    )(s_parts, g, dinv, b.reshape(1, N_CLS), w2)


def kernel(x, edge_index, W1, b1, W2, b2):
    src = edge_index[0]
    dst = edge_index[1]
    srcr = src.reshape(2, 16, NBLK, KB)
    dstr = dst.reshape(2, 16, NBLK, KB)

    deg_parts = _sc_deg(dst)
    dinv = _dinv_tc(deg_parts)[:N]
    g1 = _tc1(x, W1, dinv)
    s1p = _sc_scatter_128(srcr, dstr, g1, jnp.zeros((APAD, D_HID), jnp.float32))

    gz = _tc2(s1p, g1, dinv, b1)
    s2p = _sc_scatter_128(srcr, dstr, gz, jnp.zeros((APAD, D_HID), jnp.float32))
    return _tc3(s2p, gz, dinv, b2, W2)


# speedup vs baseline: 1.0536x; 1.0318x over previous
import functools

import jax
import jax.numpy as jnp
from jax import lax
from jax.experimental import pallas as pl
from jax.experimental.pallas import tpu as pltpu
from jax.experimental.pallas import tpu_sc as plsc

N = 10000
NPAD = 10240
E = 320000
D_IN = 128
D_HID = 128
N_CLS = 8
N_CLS_PAD = 16

NW = 32
EPT = E // NW
KB = 125
NBLK = EPT // KB
CH = 16
NCH = NBLK // CH
APAD = 10112
TRASH = APAD - 1
RPT = APAD // 16

_ROWS = 1024
_GRID = (N + _ROWS - 1) // _ROWS

_mesh = plsc.VectorSubcoreMesh(core_axis_name="c", subcore_axis_name="s")
_sc_params = pltpu.CompilerParams(needs_layout_passes=False)


@functools.partial(
    pl.kernel,
    mesh=_mesh,
    out_type=jax.ShapeDtypeStruct((NW, NPAD), jnp.float32),
    compiler_params=_sc_params,
    scratch_types=[
        pltpu.VMEM((EPT,), jnp.int32),
        pltpu.VMEM((NPAD,), jnp.float32),
    ],
)
def _sc_deg(dst_hbm, out_hbm, dst_v, deg_v):
    c = lax.axis_index("c")
    s = lax.axis_index("s")
    w = c * 16 + s
    zero16 = jnp.zeros((16,), jnp.float32)

    def zbody(i, carry):
        deg_v[pl.ds(i * 16, 16)] = zero16
        return carry

    lax.fori_loop(0, NPAD // 16, zbody, 0)
    pltpu.sync_copy(dst_hbm.at[pl.ds(w * EPT, EPT)], dst_v)
    one16 = jnp.ones((16,), jnp.float32)

    def body(i, carry):
        idx = dst_v[pl.ds(i * 16, 16)]
        plsc.addupdate_scatter(deg_v, [idx], one16)
        return carry

    lax.fori_loop(0, EPT // 16, body, 0)
    pltpu.sync_copy(deg_v, out_hbm.at[w])


def _sc_scatter_body(src_hbm, dst_hbm, g_hbm, zeros_hbm, out_hbm,
                     src_ca, dst_ca, src_cb, dst_cb, stage_a, stage_b, acc,
                     gs_a, gs_b, ss_a, ss_b, is_a, is_b):
    c = lax.axis_index("c")
    s = lax.axis_index("s")
    pltpu.sync_copy(zeros_hbm.at[pl.ds(s * RPT, RPT)],
                    acc.at[pl.ds(s * RPT, RPT)])
    plsc.subcore_barrier()

    def load_chunk(ci, src_c, dst_c, sem):
        pltpu.async_copy(src_hbm.at[c, s, pl.ds(ci * CH, CH)], src_c, sem)
        pltpu.async_copy(dst_hbm.at[c, s, pl.ds(ci * CH, CH)], dst_c, sem)

    def wait_chunk(ci, src_c, dst_c, sem):
        pltpu.make_async_copy(
            src_hbm.at[c, s, pl.ds(ci * CH, CH)], src_c, sem).wait()
        pltpu.make_async_copy(
            dst_hbm.at[c, s, pl.ds(ci * CH, CH)], dst_c, sem).wait()

    def gather(src_c, j, stage, sem):
        pltpu.async_copy(g_hbm.at[src_c.at[j]], stage, sem)

    def wait_gather(src_c, j, stage, sem):
        pltpu.make_async_copy(g_hbm.at[src_c.at[j]], stage, sem).wait()

    def scat(dst_c, j, stage, sem):
        pltpu.async_copy(stage, acc.at[dst_c.at[j]], sem, add=True)

    def wait_scat(dst_c, j, stage, sem):
        pltpu.make_async_copy(stage, acc.at[dst_c.at[j]], sem).wait()

    def process_chunk(src_c, dst_c):
        gather(src_c, 0, stage_a, gs_a)

        def body(jj, carry):
            j0 = 2 * jj
            j1 = j0 + 1
            gather(src_c, j1, stage_b, gs_b)
            wait_gather(src_c, j0, stage_a, gs_a)
            scat(dst_c, j0, stage_a, ss_a)
            wait_scat(dst_c, j0, stage_a, ss_a)
            gather(src_c, j0 + 2, stage_a, gs_a)
            wait_gather(src_c, j1, stage_b, gs_b)
            scat(dst_c, j1, stage_b, ss_b)
            wait_scat(dst_c, j1, stage_b, ss_b)
            return carry

        lax.fori_loop(0, CH // 2 - 1, body, 0)
        gather(src_c, CH - 1, stage_b, gs_b)
        wait_gather(src_c, CH - 2, stage_a, gs_a)
        scat(dst_c, CH - 2, stage_a, ss_a)
        wait_scat(dst_c, CH - 2, stage_a, ss_a)
        wait_gather(src_c, CH - 1, stage_b, gs_b)
        scat(dst_c, CH - 1, stage_b, ss_b)
        wait_scat(dst_c, CH - 1, stage_b, ss_b)

    banks = [(src_ca, dst_ca, is_a), (src_cb, dst_cb, is_b)]
    load_chunk(0, *banks[0])
    for ci in range(NCH):
        bk = banks[ci % 2]
        if ci + 1 < NCH:
            load_chunk(ci + 1, *banks[(ci + 1) % 2])
        wait_chunk(ci, *bk)
        process_chunk(bk[0], bk[1])

    plsc.subcore_barrier()
    pltpu.sync_copy(acc.at[pl.ds(s * RPT, RPT)],
                    out_hbm.at[c, pl.ds(s * RPT, RPT)])


def _make_sc_scatter(d):
    return functools.partial(
        pl.kernel,
        mesh=_mesh,
        out_type=jax.ShapeDtypeStruct((2, APAD, d), jnp.float32),
        compiler_params=_sc_params,
        scratch_types=[
            pltpu.VMEM((CH, KB), jnp.int32),
            pltpu.VMEM((CH, KB), jnp.int32),
            pltpu.VMEM((CH, KB), jnp.int32),
            pltpu.VMEM((CH, KB), jnp.int32),
            pltpu.VMEM((KB, d), jnp.float32),
            pltpu.VMEM((KB, d), jnp.float32),
            pltpu.VMEM_SHARED((APAD, d), jnp.float32),
            pltpu.SemaphoreType.DMA,
            pltpu.SemaphoreType.DMA,
            pltpu.SemaphoreType.DMA,
            pltpu.SemaphoreType.DMA,
            pltpu.SemaphoreType.DMA,
            pltpu.SemaphoreType.DMA,
        ],
    )(_sc_scatter_body)


_sc_scatter_128 = _make_sc_scatter(D_HID)


def _k1_body(degp_ref, x_ref, w_ref, g_ref, dinv_ref):
    deg = jnp.sum(degp_ref[...], axis=0) + 1.0
    dinv = lax.rsqrt(deg)[:, None]
    h = jnp.dot(x_ref[...], w_ref[...], preferred_element_type=jnp.float32)
    g_ref[...] = h * dinv
    dinv_ref[...] = dinv


def _tc1(deg_parts, x, w):
    return pl.pallas_call(
        _k1_body,
        grid=(_GRID,),
        in_specs=[pl.BlockSpec((NW, _ROWS), lambda i: (0, i)),
                  _rows_spec(D_IN), _full_spec(w.shape)],
        out_specs=[_rows_spec(w.shape[1]), _rows_spec(1)],
        out_shape=[jax.ShapeDtypeStruct((N, w.shape[1]), jnp.float32),
                   jax.ShapeDtypeStruct((N, 1), jnp.float32)],
    )(deg_parts, x, w)


def _k2_body(s_ref, g_ref, dinv_ref, b_ref, gz_ref):
    agg = s_ref[0] + s_ref[1] + g_ref[...]
    z = jnp.maximum(agg * dinv_ref[...] + b_ref[...], 0.0)
    gz_ref[...] = z * dinv_ref[...]


def _k3_body(s_ref, g_ref, dinv_ref, b_ref, w2_ref, out_ref):
    agg = (s_ref[0] + s_ref[1] + g_ref[...]) * dinv_ref[...]
    out_ref[...] = (
        jnp.dot(agg, w2_ref[...], preferred_element_type=jnp.float32)
        + b_ref[...]
    )


def _rows_spec(d):
    return pl.BlockSpec((_ROWS, d), lambda i: (i, 0))


def _parts_spec(d):
    return pl.BlockSpec((2, _ROWS, d), lambda i: (0, i, 0))


def _full_spec(shape):
    return pl.BlockSpec(shape, lambda i: (0,) * len(shape))


def _tc2(s_parts, g, dinv, b):
    d = g.shape[1]
    return pl.pallas_call(
        _k2_body,
        grid=(_GRID,),
        in_specs=[_parts_spec(d), _rows_spec(d), _rows_spec(1),
                  _full_spec((1, d))],
        out_specs=_rows_spec(d),
        out_shape=jax.ShapeDtypeStruct((N, d), jnp.float32),
    )(s_parts, g, dinv, b.reshape(1, d))


def _tc3(s_parts, g, dinv, b, w2):
    d = g.shape[1]
    return pl.pallas_call(
        _k3_body,
        grid=(_GRID,),
        in_specs=[_parts_spec(d), _rows_spec(d), _rows_spec(1),
                  _full_spec((1, N_CLS)), _full_spec(w2.shape)],
        out_specs=pl.BlockSpec((_ROWS, N_CLS), lambda i: (i, 0)),
        out_shape=jax.ShapeDtypeStruct((N, N_CLS), jnp.float32),
    )(s_parts, g, dinv, b.reshape(1, N_CLS), w2)


def kernel(x, edge_index, W1, b1, W2, b2):
    src = edge_index[0]
    dst = edge_index[1]
    srcr = src.reshape(2, 16, NBLK, KB)
    dstr = dst.reshape(2, 16, NBLK, KB)

    deg_parts = _sc_deg(dst)
    g1, dinv = _tc1(deg_parts, x, W1)
    s1p = _sc_scatter_128(srcr, dstr, g1, jnp.zeros((APAD, D_HID), jnp.float32))

    gz = _tc2(s1p, g1, dinv, b1)
    s2p = _sc_scatter_128(srcr, dstr, gz, jnp.zeros((APAD, D_HID), jnp.float32))
    return _tc3(s2p, gz, dinv, b2, W2)


# speedup vs baseline: 1.0601x; 1.0062x over previous
import functools

import jax
import jax.numpy as jnp
from jax import lax
from jax.experimental import pallas as pl
from jax.experimental.pallas import tpu as pltpu
from jax.experimental.pallas import tpu_sc as plsc

N = 10000
NPAD = 10240
E = 320000
D_IN = 128
D_HID = 128
N_CLS = 8
N_CLS_PAD = 16

NW = 32
EPT = E // NW
KB = 125
NBLK = EPT // KB
CH = 16
NCH = NBLK // CH
APAD = 10112
TRASH = APAD - 1
RPT = APAD // 16

_ROWS = 1024
_GRID = (N + _ROWS - 1) // _ROWS

_mesh = plsc.VectorSubcoreMesh(core_axis_name="c", subcore_axis_name="s")
_sc_params = pltpu.CompilerParams(needs_layout_passes=False)


@functools.partial(
    pl.kernel,
    mesh=_mesh,
    out_type=jax.ShapeDtypeStruct((NW, NPAD), jnp.float32),
    compiler_params=_sc_params,
    scratch_types=[
        pltpu.VMEM((EPT,), jnp.int32),
        pltpu.VMEM((NPAD,), jnp.float32),
    ],
)
def _sc_deg(dst_hbm, out_hbm, dst_v, deg_v):
    c = lax.axis_index("c")
    s = lax.axis_index("s")
    w = c * 16 + s
    zero16 = jnp.zeros((16,), jnp.float32)

    def zbody(i, carry):
        deg_v[pl.ds(i * 16, 16)] = zero16
        return carry

    lax.fori_loop(0, NPAD // 16, zbody, 0)
    pltpu.sync_copy(dst_hbm.at[pl.ds(w * EPT, EPT)], dst_v)
    one16 = jnp.ones((16,), jnp.float32)

    def body(i, carry):
        idx = dst_v[pl.ds(i * 16, 16)]
        plsc.addupdate_scatter(deg_v, [idx], one16)
        return carry

    lax.fori_loop(0, EPT // 16, body, 0)
    pltpu.sync_copy(deg_v, out_hbm.at[w])


def _sc_scatter_body(src_hbm, dst_hbm, g_hbm, zeros_hbm, out_hbm,
                     src_ca, dst_ca, src_cb, dst_cb, stage_a, stage_b, acc,
                     gs_a, gs_b, ss_a, ss_b, is_a, is_b):
    c = lax.axis_index("c")
    s = lax.axis_index("s")

    def load_chunk(ci, src_c, dst_c, sem):
        pltpu.async_copy(src_hbm.at[c, s, pl.ds(ci * CH, CH)], src_c, sem)
        pltpu.async_copy(dst_hbm.at[c, s, pl.ds(ci * CH, CH)], dst_c, sem)

    def wait_chunk(ci, src_c, dst_c, sem):
        pltpu.make_async_copy(
            src_hbm.at[c, s, pl.ds(ci * CH, CH)], src_c, sem).wait()
        pltpu.make_async_copy(
            dst_hbm.at[c, s, pl.ds(ci * CH, CH)], dst_c, sem).wait()

    def gather(src_c, j, stage, sem):
        pltpu.async_copy(g_hbm.at[src_c.at[j]], stage, sem)

    def wait_gather(src_c, j, stage, sem):
        pltpu.make_async_copy(g_hbm.at[src_c.at[j]], stage, sem).wait()

    def scat(dst_c, j, stage, sem):
        pltpu.async_copy(stage, acc.at[dst_c.at[j]], sem, add=True)

    def wait_scat(dst_c, j, stage, sem):
        pltpu.make_async_copy(stage, acc.at[dst_c.at[j]], sem).wait()

    def process_chunk(src_c, dst_c):
        gather(src_c, 0, stage_a, gs_a)

        def body(jj, carry):
            j0 = 2 * jj
            j1 = j0 + 1
            gather(src_c, j1, stage_b, gs_b)
            wait_gather(src_c, j0, stage_a, gs_a)
            scat(dst_c, j0, stage_a, ss_a)
            wait_scat(dst_c, j0, stage_a, ss_a)
            gather(src_c, j0 + 2, stage_a, gs_a)
            wait_gather(src_c, j1, stage_b, gs_b)
            scat(dst_c, j1, stage_b, ss_b)
            wait_scat(dst_c, j1, stage_b, ss_b)
            return carry

        lax.fori_loop(0, CH // 2 - 1, body, 0)
        gather(src_c, CH - 1, stage_b, gs_b)
        wait_gather(src_c, CH - 2, stage_a, gs_a)
        scat(dst_c, CH - 2, stage_a, ss_a)
        wait_scat(dst_c, CH - 2, stage_a, ss_a)
        wait_gather(src_c, CH - 1, stage_b, gs_b)
        scat(dst_c, CH - 1, stage_b, ss_b)
        wait_scat(dst_c, CH - 1, stage_b, ss_b)

    banks = [(src_ca, dst_ca, is_a), (src_cb, dst_cb, is_b)]
    load_chunk(0, *banks[0])
    zdesc = pltpu.async_copy(zeros_hbm.at[pl.ds(s * RPT, RPT)],
                             acc.at[pl.ds(s * RPT, RPT)], ss_a)
    zdesc.wait()
    plsc.subcore_barrier()
    for ci in range(NCH):
        bk = banks[ci % 2]
        if ci + 1 < NCH:
            load_chunk(ci + 1, *banks[(ci + 1) % 2])
        wait_chunk(ci, *bk)
        process_chunk(bk[0], bk[1])

    plsc.subcore_barrier()
    pltpu.sync_copy(acc.at[pl.ds(s * RPT, RPT)],
                    out_hbm.at[c, pl.ds(s * RPT, RPT)])


def _make_sc_scatter(d):
    return functools.partial(
        pl.kernel,
        mesh=_mesh,
        out_type=jax.ShapeDtypeStruct((2, APAD, d), jnp.float32),
        compiler_params=_sc_params,
        scratch_types=[
            pltpu.VMEM((CH, KB), jnp.int32),
            pltpu.VMEM((CH, KB), jnp.int32),
            pltpu.VMEM((CH, KB), jnp.int32),
            pltpu.VMEM((CH, KB), jnp.int32),
            pltpu.VMEM((KB, d), jnp.float32),
            pltpu.VMEM((KB, d), jnp.float32),
            pltpu.VMEM_SHARED((APAD, d), jnp.float32),
            pltpu.SemaphoreType.DMA,
            pltpu.SemaphoreType.DMA,
            pltpu.SemaphoreType.DMA,
            pltpu.SemaphoreType.DMA,
            pltpu.SemaphoreType.DMA,
            pltpu.SemaphoreType.DMA,
        ],
    )(_sc_scatter_body)


_sc_scatter_128 = _make_sc_scatter(D_HID)


def _k1_body(degp_ref, x_ref, w_ref, g_ref, dinv_ref):
    deg = jnp.sum(degp_ref[...], axis=0) + 1.0
    dinv = lax.rsqrt(deg)[:, None]
    h = jnp.dot(x_ref[...], w_ref[...], preferred_element_type=jnp.float32)
    g_ref[...] = h * dinv
    dinv_ref[...] = dinv


def _tc1(deg_parts, x, w):
    return pl.pallas_call(
        _k1_body,
        grid=(_GRID,),
        in_specs=[pl.BlockSpec((NW, _ROWS), lambda i: (0, i)),
                  _rows_spec(D_IN), _full_spec(w.shape)],
        out_specs=[_rows_spec(w.shape[1]), _rows_spec(1)],
        out_shape=[jax.ShapeDtypeStruct((N, w.shape[1]), jnp.float32),
                   jax.ShapeDtypeStruct((N, 1), jnp.float32)],
    )(deg_parts, x, w)


def _k2_body(s_ref, g_ref, dinv_ref, b_ref, gz_ref):
    agg = s_ref[0] + s_ref[1] + g_ref[...]
    z = jnp.maximum(agg * dinv_ref[...] + b_ref[...], 0.0)
    gz_ref[...] = z * dinv_ref[...]


def _k3_body(s_ref, g_ref, dinv_ref, b_ref, w2_ref, out_ref):
    agg = (s_ref[0] + s_ref[1] + g_ref[...]) * dinv_ref[...]
    out_ref[...] = (
        jnp.dot(agg, w2_ref[...], preferred_element_type=jnp.float32)
        + b_ref[...]
    )


def _rows_spec(d):
    return pl.BlockSpec((_ROWS, d), lambda i: (i, 0))


def _parts_spec(d):
    return pl.BlockSpec((2, _ROWS, d), lambda i: (0, i, 0))


def _full_spec(shape):
    return pl.BlockSpec(shape, lambda i: (0,) * len(shape))


def _tc2(s_parts, g, dinv, b):
    d = g.shape[1]
    return pl.pallas_call(
        _k2_body,
        grid=(_GRID,),
        in_specs=[_parts_spec(d), _rows_spec(d), _rows_spec(1),
                  _full_spec((1, d))],
        out_specs=_rows_spec(d),
        out_shape=jax.ShapeDtypeStruct((N, d), jnp.float32),
    )(s_parts, g, dinv, b.reshape(1, d))


def _tc3(s_parts, g, dinv, b, w2):
    d = g.shape[1]
    return pl.pallas_call(
        _k3_body,
        grid=(_GRID,),
        in_specs=[_parts_spec(d), _rows_spec(d), _rows_spec(1),
                  _full_spec((1, N_CLS)), _full_spec(w2.shape)],
        out_specs=pl.BlockSpec((_ROWS, N_CLS), lambda i: (i, 0)),
        out_shape=jax.ShapeDtypeStruct((N, N_CLS), jnp.float32),
    )(s_parts, g, dinv, b.reshape(1, N_CLS), w2)


def kernel(x, edge_index, W1, b1, W2, b2):
    src = edge_index[0]
    dst = edge_index[1]
    srcr = src.reshape(2, 16, NBLK, KB)
    dstr = dst.reshape(2, 16, NBLK, KB)

    deg_parts = _sc_deg(dst)
    g1, dinv = _tc1(deg_parts, x, W1)
    s1p = _sc_scatter_128(srcr, dstr, g1, jnp.zeros((APAD, D_HID), jnp.float32))

    gz = _tc2(s1p, g1, dinv, b1)
    s2p = _sc_scatter_128(srcr, dstr, gz, jnp.zeros((APAD, D_HID), jnp.float32))
    return _tc3(s2p, gz, dinv, b2, W2)


# speedup vs baseline: 1.0615x; 1.0013x over previous
import functools

import jax
import jax.numpy as jnp
from jax import lax
from jax.experimental import pallas as pl
from jax.experimental.pallas import tpu as pltpu
from jax.experimental.pallas import tpu_sc as plsc

N = 10000
NPAD = 10240
E = 320000
D_IN = 128
D_HID = 128
N_CLS = 8

NW = 32
EPT = E // NW
KB = 125
NBLK = EPT // KB
CH = 16
NCH = NBLK // CH
APAD = 10112
RPT = APAD // 16

_ROWS = 1024
_GRID = (N + _ROWS - 1) // _ROWS

_mesh = plsc.VectorSubcoreMesh(core_axis_name="c", subcore_axis_name="s")
_sc_params = pltpu.CompilerParams(needs_layout_passes=False)


@functools.partial(
    pl.kernel,
    mesh=_mesh,
    out_type=jax.ShapeDtypeStruct((NW, NPAD), jnp.float32),
    compiler_params=_sc_params,
    scratch_types=[
        pltpu.VMEM((EPT,), jnp.int32),
        pltpu.VMEM((NPAD,), jnp.float32),
    ],
)
def _sc_deg(dst_hbm, out_hbm, dst_v, deg_v):
    c = lax.axis_index("c")
    s = lax.axis_index("s")
    w = c * 16 + s
    zero16 = jnp.zeros((16,), jnp.float32)

    def zbody(i, carry):
        deg_v[pl.ds(i * 16, 16)] = zero16
        return carry

    lax.fori_loop(0, NPAD // 16, zbody, 0)
    pltpu.sync_copy(dst_hbm.at[pl.ds(w * EPT, EPT)], dst_v)
    one16 = jnp.ones((16,), jnp.float32)

    def body(i, carry):
        idx = dst_v[pl.ds(i * 16, 16)]
        plsc.addupdate_scatter(deg_v, [idx], one16)
        return carry

    lax.fori_loop(0, EPT // 16, body, 0)
    pltpu.sync_copy(deg_v, out_hbm.at[w])


def _sc_scatter_body(src_hbm, dst_hbm, g_hbm, zeros_hbm, out_hbm,
                     src_ca, dst_ca, src_cb, dst_cb, stage_a, stage_b, acc,
                     gs_a, gs_b, ss_a, ss_b, is_a, is_b):
    c = lax.axis_index("c")
    s = lax.axis_index("s")

    def load_chunk(ci, src_c, dst_c, sem):
        pltpu.async_copy(src_hbm.at[c, s, pl.ds(ci * CH, CH)], src_c, sem)
        pltpu.async_copy(dst_hbm.at[c, s, pl.ds(ci * CH, CH)], dst_c, sem)

    def wait_chunk(ci, src_c, dst_c, sem):
        pltpu.make_async_copy(
            src_hbm.at[c, s, pl.ds(ci * CH, CH)], src_c, sem).wait()
        pltpu.make_async_copy(
            dst_hbm.at[c, s, pl.ds(ci * CH, CH)], dst_c, sem).wait()

    def gather(src_c, j, stage, sem):
        pltpu.async_copy(g_hbm.at[src_c.at[j]], stage, sem)

    def wait_gather(src_c, j, stage, sem):
        pltpu.make_async_copy(g_hbm.at[src_c.at[j]], stage, sem).wait()

    def scat(dst_c, j, stage, sem):
        pltpu.async_copy(stage, acc.at[dst_c.at[j]], sem, add=True)

    def wait_scat(dst_c, j, stage, sem):
        pltpu.make_async_copy(stage, acc.at[dst_c.at[j]], sem).wait()

    def process_chunk(src_c, dst_c):
        gather(src_c, 0, stage_a, gs_a)

        def body(jj, carry):
            j0 = 2 * jj
            j1 = j0 + 1
            gather(src_c, j1, stage_b, gs_b)
            wait_gather(src_c, j0, stage_a, gs_a)
            scat(dst_c, j0, stage_a, ss_a)
            wait_scat(dst_c, j0, stage_a, ss_a)
            gather(src_c, j0 + 2, stage_a, gs_a)
            wait_gather(src_c, j1, stage_b, gs_b)
            scat(dst_c, j1, stage_b, ss_b)
            wait_scat(dst_c, j1, stage_b, ss_b)
            return carry

        lax.fori_loop(0, CH // 2 - 1, body, 0)
        gather(src_c, CH - 1, stage_b, gs_b)
        wait_gather(src_c, CH - 2, stage_a, gs_a)
        scat(dst_c, CH - 2, stage_a, ss_a)
        wait_scat(dst_c, CH - 2, stage_a, ss_a)
        wait_gather(src_c, CH - 1, stage_b, gs_b)
        scat(dst_c, CH - 1, stage_b, ss_b)
        wait_scat(dst_c, CH - 1, stage_b, ss_b)

    banks = [(src_ca, dst_ca, is_a), (src_cb, dst_cb, is_b)]
    load_chunk(0, *banks[0])
    zdesc = pltpu.async_copy(zeros_hbm.at[pl.ds(s * RPT, RPT)],
                             acc.at[pl.ds(s * RPT, RPT)], ss_a)
    zdesc.wait()
    plsc.subcore_barrier()
    for ci in range(NCH):
        bk = banks[ci % 2]
        if ci + 1 < NCH:
            load_chunk(ci + 1, *banks[(ci + 1) % 2])
        wait_chunk(ci, *bk)
        process_chunk(bk[0], bk[1])

    plsc.subcore_barrier()
    pltpu.sync_copy(acc.at[pl.ds(s * RPT, RPT)],
                    out_hbm.at[c, pl.ds(s * RPT, RPT)])


def _make_sc_scatter(d):
    return functools.partial(
        pl.kernel,
        mesh=_mesh,
        out_type=jax.ShapeDtypeStruct((2, APAD, d), jnp.float32),
        compiler_params=_sc_params,
        scratch_types=[
            pltpu.VMEM((CH, KB), jnp.int32),
            pltpu.VMEM((CH, KB), jnp.int32),
            pltpu.VMEM((CH, KB), jnp.int32),
            pltpu.VMEM((CH, KB), jnp.int32),
            pltpu.VMEM((KB, d), jnp.float32),
            pltpu.VMEM((KB, d), jnp.float32),
            pltpu.VMEM_SHARED((APAD, d), jnp.float32),
            pltpu.SemaphoreType.DMA,
            pltpu.SemaphoreType.DMA,
            pltpu.SemaphoreType.DMA,
            pltpu.SemaphoreType.DMA,
            pltpu.SemaphoreType.DMA,
            pltpu.SemaphoreType.DMA,
        ],
    )(_sc_scatter_body)


_sc_scatter_128 = _make_sc_scatter(D_HID)


def _k1_body(degp_ref, x_ref, w_ref, g_ref, dinv_ref):
    deg = jnp.sum(degp_ref[...], axis=0) + 1.0
    dinv = lax.rsqrt(deg)[:, None]
    h = jnp.dot(x_ref[...], w_ref[...], preferred_element_type=jnp.float32)
    g_ref[...] = h * dinv
    dinv_ref[...] = dinv


def _tc1(deg_parts, x, w):
    return pl.pallas_call(
        _k1_body,
        grid=(_GRID,),
        in_specs=[pl.BlockSpec((NW, _ROWS), lambda i: (0, i)),
                  _rows_spec(D_IN), _full_spec(w.shape)],
        out_specs=[_rows_spec(w.shape[1]), _rows_spec(1)],
        out_shape=[jax.ShapeDtypeStruct((N, w.shape[1]), jnp.float32),
                   jax.ShapeDtypeStruct((N, 1), jnp.float32)],
    )(deg_parts, x, w)


def _k2_body(s_ref, g_ref, dinv_ref, b_ref, gz_ref):
    agg = s_ref[0] + s_ref[1] + g_ref[...]
    z = jnp.maximum(agg * dinv_ref[...] + b_ref[...], 0.0)
    gz_ref[...] = z * dinv_ref[...]


def _k3_body(s_ref, g_ref, dinv_ref, b_ref, w2_ref, out_ref):
    agg = (s_ref[0] + s_ref[1] + g_ref[...]) * dinv_ref[...]
    out_ref[...] = (
        jnp.dot(agg, w2_ref[...], preferred_element_type=jnp.float32)
        + b_ref[...]
    )


def _rows_spec(d):
    return pl.BlockSpec((_ROWS, d), lambda i: (i, 0))


def _parts_spec(d):
    return pl.BlockSpec((2, _ROWS, d), lambda i: (0, i, 0))


def _full_spec(shape):
    return pl.BlockSpec(shape, lambda i: (0,) * len(shape))


def _tc2(s_parts, g, dinv, b):
    d = g.shape[1]
    return pl.pallas_call(
        _k2_body,
        grid=(_GRID,),
        in_specs=[_parts_spec(d), _rows_spec(d), _rows_spec(1),
                  _full_spec((1, d))],
        out_specs=_rows_spec(d),
        out_shape=jax.ShapeDtypeStruct((N, d), jnp.float32),
    )(s_parts, g, dinv, b.reshape(1, d))


def _tc3(s_parts, g, dinv, b, w2):
    d = g.shape[1]
    return pl.pallas_call(
        _k3_body,
        grid=(_GRID,),
        in_specs=[_parts_spec(d), _rows_spec(d), _rows_spec(1),
                  _full_spec((1, N_CLS)), _full_spec(w2.shape)],
        out_specs=pl.BlockSpec((_ROWS, N_CLS), lambda i: (i, 0)),
        out_shape=jax.ShapeDtypeStruct((N, N_CLS), jnp.float32),
    )(s_parts, g, dinv, b.reshape(1, N_CLS), w2)


def kernel(x, edge_index, W1, b1, W2, b2):
    src = edge_index[0]
    dst = edge_index[1]
    srcr = src.reshape(2, 16, NBLK, KB)
    dstr = dst.reshape(2, 16, NBLK, KB)

    deg_parts = _sc_deg(dst)
    g1, dinv = _tc1(deg_parts, x, W1)
    s1p = _sc_scatter_128(srcr, dstr, g1, jnp.zeros((APAD, D_HID), jnp.float32))

    gz = _tc2(s1p, g1, dinv, b1)
    s2p = _sc_scatter_128(srcr, dstr, gz, jnp.zeros((APAD, D_HID), jnp.float32))
    return _tc3(s2p, gz, dinv, b2, W2)
